# baseline (speedup 1.0000x reference)
import dataclasses

import jax
import jax.numpy as jnp
from jax import lax
from jax.experimental import pallas as pl
from jax.experimental.pallas import tpu as pltpu
from jax.experimental.pallas import tpu_sc as plsc

D = 256
LANES = 16
NC, NS = 2, 16
NW = NC * NS
CHUNK = 32
NBUF = 4


def _sc_gather_sq(p_hbm, q_hbm, table_hbm, out_hbm,
                  idx_p, idx_q, rows_p, rows_q, sq_v, sem_p, sem_q, sem_o):
    b_per_w = idx_p.shape[0]
    n_chunks = b_per_w // CHUNK
    wid = lax.axis_index("s") * NC + lax.axis_index("c")
    base = wid * b_per_w
    pltpu.sync_copy(p_hbm.at[pl.ds(base, b_per_w)], idx_p)
    pltpu.sync_copy(q_hbm.at[pl.ds(base, b_per_w)], idx_q)

    def issue_gather(c, b):
        off = pl.multiple_of(c * CHUNK, CHUNK)
        pltpu.async_copy(table_hbm.at[idx_p.at[pl.ds(off, CHUNK)]],
                         rows_p.at[b], sem_p.at[b])
        pltpu.async_copy(table_hbm.at[idx_q.at[pl.ds(off, CHUNK)]],
                         rows_q.at[b], sem_q.at[b])

    def wait_gather(b):
        pltpu.make_async_copy(table_hbm.at[idx_p.at[pl.ds(0, CHUNK)]],
                              rows_p.at[b], sem_p.at[b]).wait()
        pltpu.make_async_copy(table_hbm.at[idx_q.at[pl.ds(0, CHUNK)]],
                              rows_q.at[b], sem_q.at[b]).wait()

    def wait_store(b):
        pltpu.make_async_copy(sq_v.at[b], out_hbm.at[pl.ds(0, CHUNK)],
                              sem_o.at[b]).wait()

    for b in range(NBUF):
        issue_gather(b, b)

    @pl.loop(0, n_chunks, step=NBUF)
    def _outer(g):
        for b in range(NBUF):
            c = g + b
            wait_gather(b)

            @pl.when(c >= NBUF)
            def _():
                wait_store(b)

            @pl.loop(0, CHUNK)
            def _row(r):
                for k in range(D // (2 * LANES)):
                    sl = pl.ds(k * LANES, LANES)
                    vp = plsc.bitcast(rows_p[b, r, sl], jnp.bfloat16)
                    vq = plsc.bitcast(rows_q[b, r, sl], jnp.bfloat16)
                    dlt = vp - vq
                    lo, hi = plsc.unpack(dlt, format=plsc.PackFormat.INTERLEAVED)
                    slo = pl.ds(k * 2 * LANES, LANES)
                    shi = pl.ds(k * 2 * LANES + LANES, LANES)
                    sq_v[b, r, slo] = lo * lo
                    sq_v[b, r, shi] = hi * hi

            off = pl.multiple_of(c * CHUNK, CHUNK)
            pltpu.async_copy(sq_v.at[b], out_hbm.at[pl.ds(base + off, CHUNK)],
                             sem_o.at[b])

            @pl.when(c + NBUF < n_chunks)
            def _():
                issue_gather(c + NBUF, b)

    for b in range(NBUF):
        wait_store(b)


def _mlp_block(sq_ref, w1_ref, b1_ref, w2_ref, b2_ref, out_ref):
    x = sq_ref[...].astype(jnp.bfloat16)
    h = lax.dot_general(w1_ref[...], x, (((0,), (1,)), ((), ())),
                        preferred_element_type=jnp.float32)
    h = jnp.maximum(h + b1_ref[...], 0.0)
    y = lax.dot_general(w2_ref[...], h.astype(jnp.bfloat16),
                        (((0,), (0,)), ((), ())),
                        preferred_element_type=jnp.float32)
    out_ref[...] = y + b2_ref[0, 0]


def kernel(p_vertices, q_vertices, embds, W1, b1, W2, b2):
    B = p_vertices.shape[0]
    b_per_w = B // NW

    cp = pltpu.CompilerParams()
    if "needs_layout_passes" in pltpu.CompilerParams.__dataclass_fields__:
        cp = dataclasses.replace(cp, needs_layout_passes=False)
    mesh = plsc.VectorSubcoreMesh(core_axis_name="c", subcore_axis_name="s")
    sq = pl.kernel(
        _sc_gather_sq,
        out_type=jax.ShapeDtypeStruct((B, D), jnp.float32),
        mesh=mesh,
        scratch_types=[
            pltpu.VMEM((b_per_w,), jnp.int32),
            pltpu.VMEM((b_per_w,), jnp.int32),
            pltpu.VMEM((NBUF, CHUNK, D // 2), jnp.int32),
            pltpu.VMEM((NBUF, CHUNK, D // 2), jnp.int32),
            pltpu.VMEM((NBUF, CHUNK, D), jnp.float32),
            pltpu.SemaphoreType.DMA((NBUF,)),
            pltpu.SemaphoreType.DMA((NBUF,)),
            pltpu.SemaphoreType.DMA((NBUF,)),
        ],
        compiler_params=cp,
    )(p_vertices.astype(jnp.int32), q_vertices.astype(jnp.int32),
      lax.bitcast_convert_type(
          embds.astype(jnp.bfloat16).reshape(-1, D // 2, 2), jnp.int32))

    perm = (jnp.arange(D).reshape(D // (2 * LANES), LANES, 2)
            .transpose(0, 2, 1).reshape(D))
    W1p = W1[perm].astype(jnp.bfloat16)

    BM = 8192
    out = pl.pallas_call(
        _mlp_block,
        grid=(B // BM,),
        in_specs=[
            pl.BlockSpec((BM, D), lambda i: (i, 0)),
            pl.BlockSpec((D, D), lambda i: (0, 0)),
            pl.BlockSpec((D, 1), lambda i: (0, 0)),
            pl.BlockSpec((D, 1), lambda i: (0, 0)),
            pl.BlockSpec((1, 1), lambda i: (0, 0)),
        ],
        out_specs=pl.BlockSpec((1, BM), lambda i: (0, i)),
        out_shape=jax.ShapeDtypeStruct((1, B), jnp.float32),
    )(sq, W1p, b1.reshape(D, 1),
      W2.astype(jnp.bfloat16), b2.reshape(1, 1))
    return out.reshape(B)


# speedup vs baseline: 4.0551x; 4.0551x over previous
import dataclasses

import jax
import jax.numpy as jnp
from jax import lax
from jax.experimental import pallas as pl
from jax.experimental.pallas import tpu as pltpu
from jax.experimental.pallas import tpu_sc as plsc

D = 256
LANES = 16
NC, NS = 2, 16
NW = NC * NS
CHUNK = 32
NBUF = 4


def _sc_gather_sq(p_hbm, q_hbm, table_hbm, out_hbm,
                  idx_p, idx_q, rows_p, rows_q, sq_v, sem_p, sem_q, sem_o):
    b_per_w = idx_p.shape[0]
    n_chunks = b_per_w // CHUNK
    wid = lax.axis_index("s") * NC + lax.axis_index("c")
    base = wid * b_per_w
    pltpu.sync_copy(p_hbm.at[pl.ds(base, b_per_w)], idx_p)
    pltpu.sync_copy(q_hbm.at[pl.ds(base, b_per_w)], idx_q)

    def issue_gather(c, b):
        off = pl.multiple_of(c * CHUNK, CHUNK)
        pltpu.async_copy(table_hbm.at[idx_p.at[pl.ds(off, CHUNK)]],
                         rows_p.at[b], sem_p.at[b])
        pltpu.async_copy(table_hbm.at[idx_q.at[pl.ds(off, CHUNK)]],
                         rows_q.at[b], sem_q.at[b])

    def wait_gather(b):
        pltpu.make_async_copy(table_hbm.at[idx_p.at[pl.ds(0, CHUNK)]],
                              rows_p.at[b], sem_p.at[b]).wait()
        pltpu.make_async_copy(table_hbm.at[idx_q.at[pl.ds(0, CHUNK)]],
                              rows_q.at[b], sem_q.at[b]).wait()

    def wait_store(b):
        pltpu.make_async_copy(sq_v.at[b], out_hbm.at[pl.ds(0, CHUNK)],
                              sem_o.at[b]).wait()

    for b in range(NBUF):
        issue_gather(b, b)

    @pl.loop(0, n_chunks, step=NBUF)
    def _outer(g):
        for b in range(NBUF):
            c = g + b
            wait_gather(b)

            @pl.when(c >= NBUF)
            def _():
                wait_store(b)

            @pl.loop(0, CHUNK)
            def _row(r):
                for k in range(D // LANES):
                    sl = pl.ds(k * LANES, LANES)
                    dlt = rows_p[b, r, sl] - rows_q[b, r, sl]
                    sq_v[b, r, sl] = dlt * dlt

            off = pl.multiple_of(c * CHUNK, CHUNK)
            pltpu.async_copy(sq_v.at[b], out_hbm.at[pl.ds(base + off, CHUNK)],
                             sem_o.at[b])

            @pl.when(c + NBUF < n_chunks)
            def _():
                issue_gather(c + NBUF, b)

    for b in range(NBUF):
        wait_store(b)


def _mlp_block(sq_ref, w1_ref, b1_ref, w2_ref, b2_ref, out_ref):
    x = sq_ref[...].astype(jnp.bfloat16)
    h = lax.dot_general(w1_ref[...], x, (((0,), (1,)), ((), ())),
                        preferred_element_type=jnp.float32)
    h = jnp.maximum(h + b1_ref[...], 0.0)
    y = lax.dot_general(w2_ref[...], h.astype(jnp.bfloat16),
                        (((0,), (0,)), ((), ())),
                        preferred_element_type=jnp.float32)
    out_ref[...] = y + b2_ref[0, 0]


def kernel(p_vertices, q_vertices, embds, W1, b1, W2, b2):
    B = p_vertices.shape[0]
    b_per_w = B // NW

    cp = pltpu.CompilerParams(use_tc_tiling_on_sc=True)
    if "needs_layout_passes" in pltpu.CompilerParams.__dataclass_fields__:
        cp = dataclasses.replace(cp, needs_layout_passes=False)
    mesh = plsc.VectorSubcoreMesh(core_axis_name="c", subcore_axis_name="s")
    sq = pl.kernel(
        _sc_gather_sq,
        out_type=jax.ShapeDtypeStruct((B, D), jnp.float32),
        mesh=mesh,
        scratch_types=[
            pltpu.VMEM((b_per_w,), jnp.int32),
            pltpu.VMEM((b_per_w,), jnp.int32),
            pltpu.VMEM((NBUF, CHUNK, D), jnp.float32),
            pltpu.VMEM((NBUF, CHUNK, D), jnp.float32),
            pltpu.VMEM((NBUF, CHUNK, D), jnp.float32),
            pltpu.SemaphoreType.DMA((NBUF,)),
            pltpu.SemaphoreType.DMA((NBUF,)),
            pltpu.SemaphoreType.DMA((NBUF,)),
        ],
        compiler_params=cp,
    )(p_vertices.astype(jnp.int32), q_vertices.astype(jnp.int32), embds)

    BM = 8192
    out = pl.pallas_call(
        _mlp_block,
        grid=(B // BM,),
        in_specs=[
            pl.BlockSpec((BM, D), lambda i: (i, 0)),
            pl.BlockSpec((D, D), lambda i: (0, 0)),
            pl.BlockSpec((D, 1), lambda i: (0, 0)),
            pl.BlockSpec((D, 1), lambda i: (0, 0)),
            pl.BlockSpec((1, 1), lambda i: (0, 0)),
        ],
        out_specs=pl.BlockSpec((1, BM), lambda i: (0, i)),
        out_shape=jax.ShapeDtypeStruct((1, B), jnp.float32),
    )(sq, W1.astype(jnp.bfloat16), b1.reshape(D, 1),
      W2.astype(jnp.bfloat16), b2.reshape(1, 1))
    return out.reshape(B)


# speedup vs baseline: 4.0715x; 1.0040x over previous
import dataclasses

import jax
import jax.numpy as jnp
from jax import lax
from jax.experimental import pallas as pl
from jax.experimental.pallas import tpu as pltpu
from jax.experimental.pallas import tpu_sc as plsc

D = 256
LANES = 16
NC, NS = 2, 16
NW = NC * NS
CHUNK = 32
NBUF = 4


def _sc_gather_sq(p_hbm, q_hbm, table_hbm, out_hbm,
                  idx_p, idx_q, rows_p, rows_q, sq_v, sem_p, sem_q, sem_o):
    b_per_w = idx_p.shape[0]
    n_chunks = b_per_w // CHUNK
    wid = lax.axis_index("s") * NC + lax.axis_index("c")
    base = wid * b_per_w
    pltpu.sync_copy(p_hbm.at[pl.ds(base, b_per_w)], idx_p)
    pltpu.sync_copy(q_hbm.at[pl.ds(base, b_per_w)], idx_q)

    def issue_gather(c, b):
        off = pl.multiple_of(c * CHUNK, CHUNK)
        pltpu.async_copy(table_hbm.at[idx_p.at[pl.ds(off, CHUNK)]],
                         rows_p.at[b], sem_p.at[b])
        pltpu.async_copy(table_hbm.at[idx_q.at[pl.ds(off, CHUNK)]],
                         rows_q.at[b], sem_q.at[b])

    def wait_gather(b):
        pltpu.make_async_copy(table_hbm.at[idx_p.at[pl.ds(0, CHUNK)]],
                              rows_p.at[b], sem_p.at[b]).wait()
        pltpu.make_async_copy(table_hbm.at[idx_q.at[pl.ds(0, CHUNK)]],
                              rows_q.at[b], sem_q.at[b]).wait()

    def wait_store(b):
        pltpu.make_async_copy(sq_v.at[b], out_hbm.at[pl.ds(0, CHUNK)],
                              sem_o.at[b]).wait()

    for b in range(NBUF):
        issue_gather(b, b)

    @pl.loop(0, n_chunks, step=NBUF)
    def _outer(g):
        for b in range(NBUF):
            c = g + b
            wait_gather(b)

            @pl.when(c >= NBUF)
            def _():
                wait_store(b)

            @pl.loop(0, CHUNK)
            def _row(r):
                for k in range(D // LANES):
                    sl = pl.ds(k * LANES, LANES)
                    dlt = rows_p[b, r, sl] - rows_q[b, r, sl]
                    sq_v[b, r, sl] = dlt * dlt

            off = pl.multiple_of(c * CHUNK, CHUNK)
            pltpu.async_copy(sq_v.at[b], out_hbm.at[pl.ds(base + off, CHUNK)],
                             sem_o.at[b])

            @pl.when(c + NBUF < n_chunks)
            def _():
                issue_gather(c + NBUF, b)

    for b in range(NBUF):
        wait_store(b)


def _mlp_block(sq_ref, w1_ref, b1_ref, w2_ref, b2_ref, out_ref):
    x = sq_ref[...].astype(jnp.bfloat16)
    h = lax.dot_general(w1_ref[...], x, (((0,), (1,)), ((), ())),
                        preferred_element_type=jnp.float32)
    h = jnp.maximum(h + b1_ref[...], 0.0)
    y = lax.dot_general(w2_ref[...], h.astype(jnp.bfloat16),
                        (((0,), (0,)), ((), ())),
                        preferred_element_type=jnp.float32)
    out_ref[...] = y + b2_ref[0, 0]


def kernel(p_vertices, q_vertices, embds, W1, b1, W2, b2):
    B = p_vertices.shape[0]
    b_per_w = B // NW

    cp = pltpu.CompilerParams(use_tc_tiling_on_sc=True)
    if "needs_layout_passes" in pltpu.CompilerParams.__dataclass_fields__:
        cp = dataclasses.replace(cp, needs_layout_passes=False)
    mesh = plsc.VectorSubcoreMesh(core_axis_name="c", subcore_axis_name="s")
    sq = pl.kernel(
        _sc_gather_sq,
        out_type=jax.ShapeDtypeStruct((B, D), jnp.float32),
        mesh=mesh,
        scratch_types=[
            pltpu.VMEM((b_per_w,), jnp.int32),
            pltpu.VMEM((b_per_w,), jnp.int32),
            pltpu.VMEM((NBUF, CHUNK, D), jnp.float32),
            pltpu.VMEM((NBUF, CHUNK, D), jnp.float32),
            pltpu.VMEM((NBUF, CHUNK, D), jnp.float32),
            pltpu.SemaphoreType.DMA((NBUF,)),
            pltpu.SemaphoreType.DMA((NBUF,)),
            pltpu.SemaphoreType.DMA((NBUF,)),
        ],
        compiler_params=cp,
    )(p_vertices.astype(jnp.int32), q_vertices.astype(jnp.int32), embds)

    BM = 16384
    out = pl.pallas_call(
        _mlp_block,
        grid=(B // BM,),
        in_specs=[
            pl.BlockSpec((BM, D), lambda i: (i, 0)),
            pl.BlockSpec((D, D), lambda i: (0, 0)),
            pl.BlockSpec((D, 1), lambda i: (0, 0)),
            pl.BlockSpec((D, 1), lambda i: (0, 0)),
            pl.BlockSpec((1, 1), lambda i: (0, 0)),
        ],
        out_specs=pl.BlockSpec((1, BM), lambda i: (0, i)),
        out_shape=jax.ShapeDtypeStruct((1, B), jnp.float32),
    )(sq, W1.astype(jnp.bfloat16), b1.reshape(D, 1),
      W2.astype(jnp.bfloat16), b2.reshape(1, 1))
    return out.reshape(B)
